# baseline (speedup 1.0000x reference)
import jax
import jax.numpy as jnp
from jax.experimental import pallas as pl
from jax.experimental.pallas import tpu as pltpu

_B, _L, _D = 512, 512, 256


def _body(mask_ref, src_ref, y_ref, z_ref, st_ref, mk_ref, yi_ref, zi_ref,
          yo_ref, zo_ref, so_ref):
    b = pl.program_id(0)

    @pl.when(b == 0)
    def _():
        so_ref[...] = jnp.where(mk_ref[...] != 0,
                                jnp.zeros_like(st_ref[...]), st_ref[...])

    m = mask_ref[b] != 0

    @pl.when(m)
    def _():
        yo_ref[...] = jnp.broadcast_to(yi_ref[...].reshape(1, 1, _D),
                                       (1, _L, _D))
        zo_ref[...] = jnp.broadcast_to(zi_ref[...].reshape(1, 1, _D),
                                       (1, _L, _D))

    @pl.when(jnp.logical_not(m))
    def _():
        yo_ref[...] = y_ref[...]
        zo_ref[...] = z_ref[...]


def kernel(y, z, steps, mask, y_init, z_init):
    B, L, D = y.shape
    mask_i32 = mask.astype(jnp.int32)
    idx = jnp.arange(B, dtype=jnp.int32)
    src = jnp.maximum(jax.lax.cummax(jnp.where(mask, -1, idx)), 0)

    steps2d = steps.reshape(1, B)
    mask2d = mask_i32.reshape(1, B)
    yi2d = y_init.reshape(1, D)
    zi2d = z_init.reshape(1, D)

    grid_spec = pltpu.PrefetchScalarGridSpec(
        num_scalar_prefetch=2,
        grid=(B,),
        in_specs=[
            pl.BlockSpec((1, L, D), lambda i, mref, sref: (sref[i], 0, 0)),
            pl.BlockSpec((1, L, D), lambda i, mref, sref: (sref[i], 0, 0)),
            pl.BlockSpec((1, B), lambda i, mref, sref: (0, 0)),
            pl.BlockSpec((1, B), lambda i, mref, sref: (0, 0)),
            pl.BlockSpec((1, D), lambda i, mref, sref: (0, 0)),
            pl.BlockSpec((1, D), lambda i, mref, sref: (0, 0)),
        ],
        out_specs=[
            pl.BlockSpec((1, L, D), lambda i, mref, sref: (i, 0, 0)),
            pl.BlockSpec((1, L, D), lambda i, mref, sref: (i, 0, 0)),
            pl.BlockSpec((1, B), lambda i, mref, sref: (0, 0)),
        ],
    )

    y_new, z_new, so = pl.pallas_call(
        _body,
        grid_spec=grid_spec,
        out_shape=[
            jax.ShapeDtypeStruct((B, L, D), y.dtype),
            jax.ShapeDtypeStruct((B, L, D), z.dtype),
            jax.ShapeDtypeStruct((1, B), steps.dtype),
        ],
        compiler_params=pltpu.CompilerParams(
            dimension_semantics=("arbitrary",),
        ),
    )(mask_i32, src, y, z, steps2d, mask2d, yi2d, zi2d)

    return (y_new, z_new, so.reshape(B))


# speedup vs baseline: 1.3483x; 1.3483x over previous
import jax
import jax.numpy as jnp
from jax.experimental import pallas as pl
from jax.experimental.pallas import tpu as pltpu

_B, _L, _D = 512, 512, 256
_G = 8


def _body(mask_sref, y_hbm, z_hbm, st_ref, mk_ref, m3_ref, yi_ref, zi_ref,
          yo_ref, zo_ref, so_ref, y_s, z_s, sems):
    b = pl.program_id(0)
    base = b * _G

    @pl.when(b == 0)
    def _():
        so_ref[...] = jnp.where(mk_ref[...] != 0,
                                jnp.zeros_like(st_ref[...]), st_ref[...])

    for j in range(_G):
        @pl.when(mask_sref[base + j] == 0)
        def _(j=j):
            pltpu.make_async_copy(y_hbm.at[pl.ds(base + j, 1)],
                                  y_s.at[pl.ds(j, 1)], sems.at[0, j]).start()
            pltpu.make_async_copy(z_hbm.at[pl.ds(base + j, 1)],
                                  z_s.at[pl.ds(j, 1)], sems.at[1, j]).start()

    for j in range(_G):
        @pl.when(mask_sref[base + j] == 0)
        def _(j=j):
            pltpu.make_async_copy(y_hbm.at[pl.ds(base + j, 1)],
                                  y_s.at[pl.ds(j, 1)], sems.at[0, j]).wait()
            pltpu.make_async_copy(z_hbm.at[pl.ds(base + j, 1)],
                                  z_s.at[pl.ds(j, 1)], sems.at[1, j]).wait()

    mb = m3_ref[...] != 0
    yo_ref[...] = jnp.where(
        mb, jnp.broadcast_to(yi_ref[...].reshape(1, 1, _D), (_G, _L, _D)),
        y_s[...])
    zo_ref[...] = jnp.where(
        mb, jnp.broadcast_to(zi_ref[...].reshape(1, 1, _D), (_G, _L, _D)),
        z_s[...])


def kernel(y, z, steps, mask, y_init, z_init):
    B, L, D = y.shape
    mask_i32 = mask.astype(jnp.int32)

    steps2d = steps.reshape(1, B)
    mask2d = mask_i32.reshape(1, B)
    mask3d = mask_i32.reshape(B, 1, 1)
    yi2d = y_init.reshape(1, D)
    zi2d = z_init.reshape(1, D)

    grid_spec = pltpu.PrefetchScalarGridSpec(
        num_scalar_prefetch=1,
        grid=(B // _G,),
        in_specs=[
            pl.BlockSpec(memory_space=pltpu.MemorySpace.HBM),
            pl.BlockSpec(memory_space=pltpu.MemorySpace.HBM),
            pl.BlockSpec((1, B), lambda i, mref: (0, 0)),
            pl.BlockSpec((1, B), lambda i, mref: (0, 0)),
            pl.BlockSpec((_G, 1, 1), lambda i, mref: (i, 0, 0)),
            pl.BlockSpec((1, D), lambda i, mref: (0, 0)),
            pl.BlockSpec((1, D), lambda i, mref: (0, 0)),
        ],
        out_specs=[
            pl.BlockSpec((_G, L, D), lambda i, mref: (i, 0, 0)),
            pl.BlockSpec((_G, L, D), lambda i, mref: (i, 0, 0)),
            pl.BlockSpec((1, B), lambda i, mref: (0, 0)),
        ],
        scratch_shapes=[
            pltpu.VMEM((_G, L, D), jnp.float32),
            pltpu.VMEM((_G, L, D), jnp.float32),
            pltpu.SemaphoreType.DMA((2, _G)),
        ],
    )

    y_new, z_new, so = pl.pallas_call(
        _body,
        grid_spec=grid_spec,
        out_shape=[
            jax.ShapeDtypeStruct((B, L, D), y.dtype),
            jax.ShapeDtypeStruct((B, L, D), z.dtype),
            jax.ShapeDtypeStruct((1, B), steps.dtype),
        ],
        compiler_params=pltpu.CompilerParams(
            dimension_semantics=("arbitrary",),
        ),
    )(mask_i32, y, z, steps2d, mask2d, mask3d, yi2d, zi2d)

    return (y_new, z_new, so.reshape(B))


# speedup vs baseline: 1.7868x; 1.3252x over previous
import jax
import jax.numpy as jnp
from jax.experimental import pallas as pl
from jax.experimental.pallas import tpu as pltpu

_B, _L, _D = 512, 512, 256
_G = 16


def _body(mask_sref, y_hbm, z_hbm, st_ref, mk_ref, yi_ref, zi_ref,
          yo_ref, zo_ref, so_ref, sems):
    b = pl.program_id(0)
    base = b * _G

    @pl.when(b == 0)
    def _():
        so_ref[...] = jnp.where(mk_ref[...] != 0,
                                jnp.zeros_like(st_ref[...]), st_ref[...])

    for j in range(_G):
        @pl.when(mask_sref[base + j] == 0)
        def _(j=j):
            pltpu.make_async_copy(y_hbm.at[pl.ds(base + j, 1)],
                                  yo_ref.at[pl.ds(j, 1)], sems.at[0, j]).start()
            pltpu.make_async_copy(z_hbm.at[pl.ds(base + j, 1)],
                                  zo_ref.at[pl.ds(j, 1)], sems.at[1, j]).start()

    yi_row = jnp.broadcast_to(yi_ref[...].reshape(1, 1, _D), (1, _L, _D))
    zi_row = jnp.broadcast_to(zi_ref[...].reshape(1, 1, _D), (1, _L, _D))
    for j in range(_G):
        @pl.when(mask_sref[base + j] != 0)
        def _(j=j):
            yo_ref[pl.ds(j, 1)] = yi_row
            zo_ref[pl.ds(j, 1)] = zi_row

    for j in range(_G):
        @pl.when(mask_sref[base + j] == 0)
        def _(j=j):
            pltpu.make_async_copy(y_hbm.at[pl.ds(base + j, 1)],
                                  yo_ref.at[pl.ds(j, 1)], sems.at[0, j]).wait()
            pltpu.make_async_copy(z_hbm.at[pl.ds(base + j, 1)],
                                  zo_ref.at[pl.ds(j, 1)], sems.at[1, j]).wait()


def kernel(y, z, steps, mask, y_init, z_init):
    B, L, D = y.shape
    mask_i32 = mask.astype(jnp.int32)

    steps2d = steps.reshape(1, B)
    mask2d = mask_i32.reshape(1, B)
    yi2d = y_init.reshape(1, D)
    zi2d = z_init.reshape(1, D)

    grid_spec = pltpu.PrefetchScalarGridSpec(
        num_scalar_prefetch=1,
        grid=(B // _G,),
        in_specs=[
            pl.BlockSpec(memory_space=pltpu.MemorySpace.HBM),
            pl.BlockSpec(memory_space=pltpu.MemorySpace.HBM),
            pl.BlockSpec((1, B), lambda i, mref: (0, 0)),
            pl.BlockSpec((1, B), lambda i, mref: (0, 0)),
            pl.BlockSpec((1, D), lambda i, mref: (0, 0)),
            pl.BlockSpec((1, D), lambda i, mref: (0, 0)),
        ],
        out_specs=[
            pl.BlockSpec((_G, L, D), lambda i, mref: (i, 0, 0)),
            pl.BlockSpec((_G, L, D), lambda i, mref: (i, 0, 0)),
            pl.BlockSpec((1, B), lambda i, mref: (0, 0)),
        ],
        scratch_shapes=[
            pltpu.SemaphoreType.DMA((2, _G)),
        ],
    )

    y_new, z_new, so = pl.pallas_call(
        _body,
        grid_spec=grid_spec,
        out_shape=[
            jax.ShapeDtypeStruct((B, L, D), y.dtype),
            jax.ShapeDtypeStruct((B, L, D), z.dtype),
            jax.ShapeDtypeStruct((1, B), steps.dtype),
        ],
        compiler_params=pltpu.CompilerParams(
            dimension_semantics=("arbitrary",),
        ),
    )(mask_i32, y, z, steps2d, mask2d, yi2d, zi2d)

    return (y_new, z_new, so.reshape(B))
